# baseline (speedup 1.0000x reference)
import functools

import jax
import jax.numpy as jnp
from jax import lax
from jax.experimental import pallas as pl
from jax.experimental.pallas import tpu as pltpu
from jax.experimental.pallas import tpu_sc as plsc

N = 10000
E = 320000
D = 16
NC, NS = 2, 16
NW = NC * NS
EPW = E // NW
CHUNK = 2000
NCHUNK = EPW // CHUNK
NPAD = 10240
DSL = NPAD // NS
RSL = NPAD // NS
NP8 = N // 8
NPAD8 = NPAD // 8
RSL8 = RSL // 8

_mesh = plsc.VectorSubcoreMesh(core_axis_name="c", subcore_axis_name="s")
_sc_params = pltpu.CompilerParams(use_tc_tiling_on_sc=False)


@functools.partial(
    pl.kernel,
    out_type=jax.ShapeDtypeStruct((NC * NPAD,), jnp.float32),
    mesh=_mesh,
    scratch_types=[
        pltpu.VMEM((EPW,), jnp.int32),
        pltpu.VMEM((EPW,), jnp.float32),
        pltpu.VMEM((DSL,), jnp.float32),
        pltpu.VMEM_SHARED((NPAD,), jnp.float32),
        pltpu.SemaphoreType.DMA,
    ],
    compiler_params=_sc_params,
)
def _deg_kernel(ei_hbm, out_hbm, idx_v, ones_v, z_v, acc_sh, sem):
    c = lax.axis_index("c")
    s = lax.axis_index("s")
    wid = s * NC + c
    idx_cp = pltpu.async_copy(ei_hbm.at[pl.ds(E + wid * EPW, EPW)], idx_v, sem)

    def fill_ones(i, carry):
        ones_v[pl.ds(i * 16, 16)] = jnp.ones((16,), jnp.float32)
        return carry

    lax.fori_loop(0, EPW // 16, fill_ones, 0)

    def fill_zero(i, carry):
        z_v[pl.ds(i * 16, 16)] = jnp.zeros((16,), jnp.float32)
        return carry

    lax.fori_loop(0, DSL // 16, fill_zero, 0)

    pltpu.sync_copy(z_v, acc_sh.at[pl.ds(s * DSL, DSL)])
    plsc.subcore_barrier()

    idx_cp.wait()
    pltpu.sync_copy(ones_v, acc_sh.at[idx_v], add=True)
    plsc.subcore_barrier()

    pltpu.sync_copy(acc_sh.at[pl.ds(s * DSL, DSL)],
                    out_hbm.at[pl.ds(c * NPAD + s * DSL, DSL)])


@functools.partial(
    pl.kernel,
    out_type=jax.ShapeDtypeStruct((NC * NPAD, D), jnp.float32),
    mesh=_mesh,
    scratch_types=[
        pltpu.VMEM((3, CHUNK), jnp.int32),
        pltpu.VMEM((3, CHUNK), jnp.int32),
        pltpu.VMEM((2, CHUNK, D), jnp.float32),
        pltpu.VMEM((RSL, D), jnp.float32),
        pltpu.VMEM_SHARED((NPAD, D), jnp.float32),
        pltpu.SemaphoreType.DMA,
        pltpu.SemaphoreType.DMA,
        pltpu.SemaphoreType.DMA,
        pltpu.SemaphoreType.DMA,
        pltpu.SemaphoreType.DMA,
        pltpu.SemaphoreType.DMA,
        pltpu.SemaphoreType.DMA,
    ],
    compiler_params=_sc_params,
)
def _agg_kernel(y_hbm, ei_hbm, out_hbm,
                si_v, di_v, rows_v, z_v, acc_sh,
                sem_i0, sem_i1, sem_i2, sem_g0, sem_g1, sem_s0, sem_s1):
    c = lax.axis_index("c")
    s = lax.axis_index("s")
    wid = s * NC + c
    sem_i = (sem_i0, sem_i1, sem_i2)
    sem_g = (sem_g0, sem_g1)
    sem_s = (sem_s0, sem_s1)

    def start_idx(k):
        b = k % 3
        base = wid * EPW + k * CHUNK
        pltpu.async_copy(ei_hbm.at[pl.ds(base, CHUNK)], si_v.at[b], sem_i[b])
        pltpu.async_copy(ei_hbm.at[pl.ds(E + base, CHUNK)], di_v.at[b], sem_i[b])

    def wait_idx(k):
        b = k % 3
        pltpu.make_async_copy(ei_hbm.at[pl.ds(0, CHUNK)], si_v.at[b],
                              sem_i[b]).wait()
        pltpu.make_async_copy(ei_hbm.at[pl.ds(0, CHUNK)], di_v.at[b],
                              sem_i[b]).wait()

    def start_gather(k):
        return pltpu.async_copy(y_hbm.at[si_v.at[k % 3]], rows_v.at[k % 2],
                                sem_g[k % 2])

    def start_scatter(k):
        return pltpu.async_copy(rows_v.at[k % 2], acc_sh.at[di_v.at[k % 3]],
                                sem_s[k % 2], add=True)

    def fill_zero(i, carry):
        z_v[i, :] = jnp.zeros((D,), jnp.float32)
        return carry

    start_idx(0)
    start_idx(1)
    start_idx(2)
    lax.fori_loop(0, RSL, fill_zero, 0)
    pltpu.sync_copy(z_v, acc_sh.at[pl.ds(s * RSL, RSL)])
    plsc.subcore_barrier()

    gath = [None, None]
    scat = [None, None]
    wait_idx(0)
    gath[0] = start_gather(0)

    for k in range(NCHUNK):
        b = k % 2
        nb = (k + 1) % 2
        if k >= 1:
            scat[nb].wait()
            if k + 2 < NCHUNK:
                start_idx(k + 2)
        gath[b].wait()
        if k + 1 < NCHUNK:
            wait_idx(k + 1)
            gath[nb] = start_gather(k + 1)
        scat[b] = start_scatter(k)

    scat[(NCHUNK - 1) % 2].wait()
    plsc.subcore_barrier()
    pltpu.sync_copy(acc_sh.at[pl.ds(s * RSL, RSL)],
                    out_hbm.at[pl.ds(c * NPAD + s * RSL, RSL)])


def _tc1a_body(x3_ref, w1b_ref, xwp_ref):
    yp = jnp.dot(x3_ref[:, 0, :], w1b_ref[0:128, :],
                 preferred_element_type=jnp.float32)
    for j in range(1, 8):
        yp = yp + jnp.dot(x3_ref[:, j, :], w1b_ref[128 * j:128 * j + 128, :],
                          preferred_element_type=jnp.float32)
    xwp_ref[...] = yp


_tc1a = pl.pallas_call(
    _tc1a_body,
    out_shape=jax.ShapeDtypeStruct((NP8, 128), jnp.float32),
)


def _tc1b_body(xwp_ref, dp3_ref, sel_ref, y1_ref, db_ref):
    deg = dp3_ref[:, :, 0] + dp3_ref[:, :, 1] + 1.0
    degb = jnp.dot(deg, sel_ref[...], preferred_element_type=jnp.float32)
    db = lax.rsqrt(degb)
    y1_ref[...] = xwp_ref[...] * db
    db_ref[...] = db


_tc1b = pl.pallas_call(
    _tc1b_body,
    out_shape=(jax.ShapeDtypeStruct((NP8, 128), jnp.float32),
               jax.ShapeDtypeStruct((NP8, 128), jnp.float32)),
)


def _tc2_body(a_ref, y1_ref, db_ref, b1_ref, w2b_ref, y2_ref):
    agg = a_ref[0:NP8, :] + a_ref[NPAD8:NPAD8 + NP8, :] + y1_ref[...]
    h = jnp.maximum(db_ref[...] * agg + b1_ref[...], 0.0)
    hw = jnp.dot(h, w2b_ref[...], preferred_element_type=jnp.float32)
    y2_ref[...] = hw * db_ref[...]


_tc2 = pl.pallas_call(
    _tc2_body,
    out_shape=jax.ShapeDtypeStruct((NP8, 128), jnp.float32),
)


def _tc3_body(a_ref, y2_ref, db_ref, b2_ref, out_ref):
    agg = a_ref[0:NP8, :] + a_ref[NPAD8:NPAD8 + NP8, :] + y2_ref[...]
    o = db_ref[...] * agg + b2_ref[...]
    for j in range(8):
        out_ref[j] = o[:, D * j:D * j + D]


_tc3 = pl.pallas_call(
    _tc3_body,
    out_shape=jax.ShapeDtypeStruct((8, NP8, D), jnp.float32),
)


def kernel(x, edge_index, W1, b1, W2, b2):
    ei = edge_index.astype(jnp.int32).reshape(2 * E)
    b1p = jnp.tile(b1.reshape(1, D), (1, 8))
    b2p = jnp.tile(b2.reshape(1, D), (1, 8))
    eye8 = jnp.eye(8, dtype=jnp.float32)
    w1b = jnp.kron(eye8, W1)
    w2b = jnp.kron(eye8, W2)
    degp = _deg_kernel(ei)
    xwp = _tc1a(x.reshape(NP8, 8, 128), w1b)
    dp3 = degp.reshape(NC, NPAD)[:, :N].T.reshape(NP8, 8, NC)
    sel = jnp.kron(eye8, jnp.ones((1, D), jnp.float32))
    y1, db = _tc1b(xwp, dp3, sel)
    a1 = _agg_kernel(y1.reshape(N, D), ei)
    y2 = _tc2(a1.reshape(NC * NPAD8, 128), y1, db, b1p, w2b)
    a2 = _agg_kernel(y2.reshape(N, D), ei)
    out3 = _tc3(a2.reshape(NC * NPAD8, 128), y2, db, b2p)
    return jnp.transpose(out3, (1, 0, 2)).reshape(N, D)


# speedup vs baseline: 1.0532x; 1.0532x over previous
import functools

import jax
import jax.numpy as jnp
from jax import lax
from jax.experimental import pallas as pl
from jax.experimental.pallas import tpu as pltpu
from jax.experimental.pallas import tpu_sc as plsc

N = 10000
E = 320000
D = 16
NC, NS = 2, 16
NW = NC * NS
EPW = E // NW
CHUNK = 2000
NCHUNK = EPW // CHUNK
NPAD = 10240
DSL = NPAD // NS
RSL = NPAD // NS
NP8 = N // 8
NPAD8 = NPAD // 8
RSL8 = RSL // 8

_mesh = plsc.VectorSubcoreMesh(core_axis_name="c", subcore_axis_name="s")
_sc_params = pltpu.CompilerParams(use_tc_tiling_on_sc=False)


@functools.partial(
    pl.kernel,
    out_type=jax.ShapeDtypeStruct((NC * NPAD,), jnp.float32),
    mesh=_mesh,
    scratch_types=[
        pltpu.VMEM((EPW,), jnp.int32),
        pltpu.VMEM((EPW,), jnp.float32),
        pltpu.VMEM((DSL,), jnp.float32),
        pltpu.VMEM_SHARED((NPAD,), jnp.float32),
        pltpu.SemaphoreType.DMA,
    ],
    compiler_params=_sc_params,
)
def _deg_kernel(ei_hbm, out_hbm, idx_v, ones_v, z_v, acc_sh, sem):
    c = lax.axis_index("c")
    s = lax.axis_index("s")
    wid = s * NC + c
    idx_cp = pltpu.async_copy(ei_hbm.at[pl.ds(E + wid * EPW, EPW)], idx_v, sem)

    def fill_ones(i, carry):
        ones_v[pl.ds(i * 16, 16)] = jnp.ones((16,), jnp.float32)
        return carry

    lax.fori_loop(0, EPW // 16, fill_ones, 0)

    def fill_zero(i, carry):
        z_v[pl.ds(i * 16, 16)] = jnp.zeros((16,), jnp.float32)
        return carry

    lax.fori_loop(0, DSL // 16, fill_zero, 0)

    pltpu.sync_copy(z_v, acc_sh.at[pl.ds(s * DSL, DSL)])
    plsc.subcore_barrier()

    idx_cp.wait()
    pltpu.sync_copy(ones_v, acc_sh.at[idx_v], add=True)
    plsc.subcore_barrier()

    pltpu.sync_copy(acc_sh.at[pl.ds(s * DSL, DSL)],
                    out_hbm.at[pl.ds(c * NPAD + s * DSL, DSL)])


@functools.partial(
    pl.kernel,
    out_type=jax.ShapeDtypeStruct((NC * NPAD, D), jnp.float32),
    mesh=_mesh,
    scratch_types=[
        pltpu.VMEM((3, CHUNK), jnp.int32),
        pltpu.VMEM((3, CHUNK), jnp.int32),
        pltpu.VMEM((2, CHUNK, D), jnp.float32),
        pltpu.VMEM((RSL, D), jnp.float32),
        pltpu.VMEM_SHARED((NPAD, D), jnp.float32),
        pltpu.SemaphoreType.DMA,
        pltpu.SemaphoreType.DMA,
        pltpu.SemaphoreType.DMA,
        pltpu.SemaphoreType.DMA,
        pltpu.SemaphoreType.DMA,
        pltpu.SemaphoreType.DMA,
        pltpu.SemaphoreType.DMA,
    ],
    compiler_params=_sc_params,
)
def _agg_kernel(y_hbm, ei_hbm, out_hbm,
                si_v, di_v, rows_v, z_v, acc_sh,
                sem_i0, sem_i1, sem_i2, sem_g0, sem_g1, sem_s0, sem_s1):
    c = lax.axis_index("c")
    s = lax.axis_index("s")
    wid = s * NC + c
    sem_i = (sem_i0, sem_i1, sem_i2)
    sem_g = (sem_g0, sem_g1)
    sem_s = (sem_s0, sem_s1)

    def start_idx(k):
        b = k % 3
        base = wid * EPW + k * CHUNK
        pltpu.async_copy(ei_hbm.at[pl.ds(base, CHUNK)], si_v.at[b], sem_i[b])
        pltpu.async_copy(ei_hbm.at[pl.ds(E + base, CHUNK)], di_v.at[b], sem_i[b])

    def wait_idx(k):
        b = k % 3
        pltpu.make_async_copy(ei_hbm.at[pl.ds(0, CHUNK)], si_v.at[b],
                              sem_i[b]).wait()
        pltpu.make_async_copy(ei_hbm.at[pl.ds(0, CHUNK)], di_v.at[b],
                              sem_i[b]).wait()

    def start_gather(k):
        return pltpu.async_copy(y_hbm.at[si_v.at[k % 3]], rows_v.at[k % 2],
                                sem_g[k % 2])

    def start_scatter(k):
        return pltpu.async_copy(rows_v.at[k % 2], acc_sh.at[di_v.at[k % 3]],
                                sem_s[k % 2], add=True)

    def fill_zero(i, carry):
        z_v[i, :] = jnp.zeros((D,), jnp.float32)
        return carry

    start_idx(0)
    start_idx(1)
    start_idx(2)
    lax.fori_loop(0, RSL, fill_zero, 0)
    pltpu.sync_copy(z_v, acc_sh.at[pl.ds(s * RSL, RSL)])
    plsc.subcore_barrier()

    gath = [None, None]
    scat = [None, None]
    wait_idx(0)
    gath[0] = start_gather(0)

    for k in range(NCHUNK):
        b = k % 2
        nb = (k + 1) % 2
        if k >= 1:
            scat[nb].wait()
            if k + 2 < NCHUNK:
                start_idx(k + 2)
        gath[b].wait()
        if k + 1 < NCHUNK:
            wait_idx(k + 1)
            gath[nb] = start_gather(k + 1)
        scat[b] = start_scatter(k)

    scat[(NCHUNK - 1) % 2].wait()
    plsc.subcore_barrier()
    pltpu.sync_copy(acc_sh.at[pl.ds(s * RSL, RSL)],
                    out_hbm.at[pl.ds(c * NPAD + s * RSL, RSL)])


def _tc1a_body(x3_ref, w1b_ref, xwp_ref):
    yp = jnp.dot(x3_ref[:, 0, :], w1b_ref[0:128, :],
                 preferred_element_type=jnp.float32)
    for j in range(1, 8):
        yp = yp + jnp.dot(x3_ref[:, j, :], w1b_ref[128 * j:128 * j + 128, :],
                          preferred_element_type=jnp.float32)
    xwp_ref[...] = yp


_tc1a = pl.pallas_call(
    _tc1a_body,
    out_shape=jax.ShapeDtypeStruct((NP8, 128), jnp.float32),
)


def _tc1b_body(xwp_ref, dp3_ref, sel_ref, y1_ref, db_ref):
    deg = dp3_ref[:, :, 0] + dp3_ref[:, :, 1] + 1.0
    degb = jnp.dot(deg, sel_ref[...], preferred_element_type=jnp.float32)
    db = lax.rsqrt(degb)
    y1_ref[...] = xwp_ref[...] * db
    db_ref[...] = db


_tc1b = pl.pallas_call(
    _tc1b_body,
    out_shape=(jax.ShapeDtypeStruct((NP8, 128), jnp.float32),
               jax.ShapeDtypeStruct((NP8, 128), jnp.float32)),
)


def _tc2_body(a_ref, y1_ref, db_ref, b1_ref, w2b_ref, y2_ref):
    agg = a_ref[0:NP8, :] + a_ref[NPAD8:NPAD8 + NP8, :] + y1_ref[...]
    h = jnp.maximum(db_ref[...] * agg + b1_ref[...], 0.0)
    hw = jnp.dot(h, w2b_ref[...], preferred_element_type=jnp.float32)
    y2_ref[...] = hw * db_ref[...]


_tc2 = pl.pallas_call(
    _tc2_body,
    out_shape=jax.ShapeDtypeStruct((NP8, 128), jnp.float32),
)


def _tc3_body(a_ref, y2_ref, db_ref, b2_ref, out_ref):
    agg = a_ref[0:NP8, :] + a_ref[NPAD8:NPAD8 + NP8, :] + y2_ref[...]
    out_ref[...] = db_ref[...] * agg + b2_ref[...]


_tc3 = pl.pallas_call(
    _tc3_body,
    out_shape=jax.ShapeDtypeStruct((NP8, 128), jnp.float32),
)


def kernel(x, edge_index, W1, b1, W2, b2):
    ei = edge_index.astype(jnp.int32).reshape(2 * E)
    b1p = jnp.tile(b1.reshape(1, D), (1, 8))
    b2p = jnp.tile(b2.reshape(1, D), (1, 8))
    eye8 = jnp.eye(8, dtype=jnp.float32)
    w1b = jnp.kron(eye8, W1)
    w2b = jnp.kron(eye8, W2)
    degp = _deg_kernel(ei)
    xwp = _tc1a(x.reshape(NP8, 8, 128), w1b)
    dp3 = degp.reshape(NC, NPAD)[:, :N].T.reshape(NP8, 8, NC)
    sel = jnp.kron(eye8, jnp.ones((1, D), jnp.float32))
    y1, db = _tc1b(xwp, dp3, sel)
    a1 = _agg_kernel(y1.reshape(N, D), ei)
    y2 = _tc2(a1.reshape(NC * NPAD8, 128), y1, db, b1p, w2b)
    a2 = _agg_kernel(y2.reshape(N, D), ei)
    return _tc3(a2.reshape(NC * NPAD8, 128), y2, db, b2p).reshape(N, D)
